# baseline (speedup 1.0000x reference)
import functools

import jax
import jax.numpy as jnp
from jax import lax
from jax.experimental import pallas as pl
from jax.experimental.pallas import tpu as pltpu
from jax.experimental.pallas import tpu_sc as plsc

_B = 4096
_L = 200
_D = 64


def _make_sc_gather():
    info = plsc.get_sparse_core_info()
    nw = info.num_cores * info.num_subcores
    b_per_w = _B // nw
    mesh = plsc.VectorSubcoreMesh(core_axis_name="c", subcore_axis_name="s")

    @functools.partial(
        pl.kernel,
        mesh=mesh,
        out_type=jax.ShapeDtypeStruct((_B, _D), jnp.float32),
        scratch_types=[
            pltpu.VMEM((b_per_w,), jnp.int32),
            pltpu.VMEM((b_per_w, _D), jnp.float32),
            pltpu.SemaphoreType.DMA,
        ],
        compiler_params=pltpu.CompilerParams(use_tc_tiling_on_sc=False),
    )
    def sc_gather(table_hbm, idx_hbm, out_hbm, idx_v, rows_v, sem):
        wid = lax.axis_index("s") * info.num_cores + lax.axis_index("c")
        base = wid * b_per_w
        pltpu.sync_copy(idx_hbm.at[pl.ds(base, b_per_w)], idx_v)
        pltpu.async_copy(table_hbm.at[idx_v], rows_v, sem).wait()
        pltpu.sync_copy(rows_v, out_hbm.at[pl.ds(base, b_per_w)])

    return sc_gather


_sc_gather = _make_sc_gather()

_N_CHUNKS = 8
_ROWS_PER_CHUNK = _L // _N_CHUNKS


def _copy_body(feat_ref, out_ref, sem):
    copies = []
    for i in range(_N_CHUNKS):
        c = pltpu.make_async_copy(
            feat_ref.at[pl.ds(i * _ROWS_PER_CHUNK, _ROWS_PER_CHUNK)],
            out_ref.at[pl.ds(1 + i * _ROWS_PER_CHUNK, _ROWS_PER_CHUNK)],
            sem,
        )
        c.start()
        copies.append(c)
    for c in copies:
        c.wait()


_copy_feat = pl.pallas_call(
    _copy_body,
    in_specs=[pl.BlockSpec(memory_space=pl.ANY)],
    out_specs=pl.BlockSpec(memory_space=pl.ANY),
    out_shape=jax.ShapeDtypeStruct((_L + 1, _D, _B), jnp.float32),
    scratch_shapes=[pltpu.SemaphoreType.DMA],
)


def _patch_body(emb_ref, prev_ref, out_ref, sem):
    del prev_ref
    c = pltpu.make_async_copy(emb_ref, out_ref.at[pl.ds(0, 1)], sem)
    c.start()
    c.wait()


_patch = pl.pallas_call(
    _patch_body,
    in_specs=[
        pl.BlockSpec(memory_space=pl.ANY),
        pl.BlockSpec(memory_space=pl.ANY),
    ],
    out_specs=pl.BlockSpec(memory_space=pl.ANY),
    out_shape=jax.ShapeDtypeStruct((_L + 1, _D, _B), jnp.float32),
    scratch_shapes=[pltpu.SemaphoreType.DMA],
    input_output_aliases={1: 0},
)


def kernel(feature, index, table):
    idx = index.reshape(-1).astype(jnp.int32)
    feat_t = jnp.transpose(feature, (1, 2, 0))
    emb = _sc_gather(table, idx)
    emb_t = jnp.transpose(emb)[None]
    out_t = _copy_feat(feat_t)
    out_t = _patch(emb_t, out_t)
    return jnp.transpose(out_t, (2, 0, 1))


# speedup vs baseline: 9.2779x; 9.2779x over previous
import functools

import jax
import jax.numpy as jnp
from jax import lax
from jax.experimental import pallas as pl
from jax.experimental.pallas import tpu as pltpu
from jax.experimental.pallas import tpu_sc as plsc

_B = 4096
_L = 200
_D = 64


def _make_sc_gather():
    info = plsc.get_sparse_core_info()
    nw = info.num_cores * info.num_subcores
    b_per_w = _B // nw
    mesh = plsc.VectorSubcoreMesh(core_axis_name="c", subcore_axis_name="s")

    @functools.partial(
        pl.kernel,
        mesh=mesh,
        out_type=jax.ShapeDtypeStruct((_B, _D), jnp.float32),
        scratch_types=[
            pltpu.VMEM((b_per_w,), jnp.int32),
            pltpu.VMEM((b_per_w, _D), jnp.float32),
            pltpu.SemaphoreType.DMA,
        ],
        compiler_params=pltpu.CompilerParams(use_tc_tiling_on_sc=False),
    )
    def sc_gather(table_hbm, idx_hbm, out_hbm, idx_v, rows_v, sem):
        wid = lax.axis_index("s") * info.num_cores + lax.axis_index("c")
        base = wid * b_per_w
        pltpu.sync_copy(idx_hbm.at[pl.ds(base, b_per_w)], idx_v)
        pltpu.async_copy(table_hbm.at[idx_v], rows_v, sem).wait()
        pltpu.sync_copy(rows_v, out_hbm.at[pl.ds(base, b_per_w)])

    return sc_gather


_sc_gather = _make_sc_gather()

_BB = 128


def _copy_body(feat_ref, out_ref):
    out_ref[1:, :, :] = feat_ref[...]


_copy_feat = pl.pallas_call(
    _copy_body,
    grid=(_B // _BB,),
    in_specs=[pl.BlockSpec((_L, _D, _BB), lambda i: (0, 0, i))],
    out_specs=pl.BlockSpec((_L + 1, _D, _BB), lambda i: (0, 0, i)),
    out_shape=jax.ShapeDtypeStruct((_L + 1, _D, _B), jnp.float32),
)


def _patch_body(emb_ref, prev_ref, out_ref):
    del prev_ref
    out_ref[...] = emb_ref[...]


_patch = pl.pallas_call(
    _patch_body,
    grid=(1,),
    in_specs=[
        pl.BlockSpec((1, _D, _B), lambda i: (0, 0, 0)),
        pl.BlockSpec(memory_space=pl.ANY),
    ],
    out_specs=pl.BlockSpec((1, _D, _B), lambda i: (0, 0, 0)),
    out_shape=jax.ShapeDtypeStruct((_L + 1, _D, _B), jnp.float32),
    input_output_aliases={1: 0},
)


def kernel(feature, index, table):
    idx = index.reshape(-1).astype(jnp.int32)
    feat_t = jnp.transpose(feature, (1, 2, 0))
    emb = _sc_gather(table, idx)
    emb_t = jnp.transpose(emb)[None]
    out_t = _copy_feat(feat_t)
    out_t = _patch(emb_t, out_t)
    return jnp.transpose(out_t, (2, 0, 1))
